# Initial kernel scaffold; baseline (speedup 1.0000x reference)
import jax
import jax.numpy as jnp
from jax.experimental import pallas as pl


def kernel(inputs):
    raise NotImplementedError("write your pallas kernel here")


# speedup vs baseline: 4.4041x; 4.4041x over previous
import functools

import jax
import jax.numpy as jnp
from jax.experimental import pallas as pl
from jax.experimental.pallas import tpu as pltpu

_BLOCK_ROWS = 512


def _sum_kernel(x_ref, acc_ref):
    i = pl.program_id(0)

    @pl.when(i == 0)
    def _init():
        acc_ref[...] = jnp.zeros_like(acc_ref)

    acc_ref[...] += jnp.sum(x_ref[...], axis=0, keepdims=True)


def _replace_kernel(x_ref, sums_ref, o_ref, *, inv_n):
    mean = sums_ref[...] * inv_n
    x = x_ref[...]
    lane = jax.lax.broadcasted_iota(jnp.int32, x.shape, dimension=1)
    o_ref[...] = jnp.where(lane % 2 == 0, jnp.broadcast_to(mean, x.shape), x)


def kernel(inputs):
    orig_shape = inputs.shape
    c = orig_shape[-1]
    rows = 1
    for d in orig_shape[:-1]:
        rows *= d
    x = inputs.reshape(rows, c)
    nblk = rows // _BLOCK_ROWS

    sums = pl.pallas_call(
        _sum_kernel,
        grid=(nblk,),
        in_specs=[pl.BlockSpec((_BLOCK_ROWS, c), lambda i: (i, 0))],
        out_specs=pl.BlockSpec((1, c), lambda i: (0, 0)),
        out_shape=jax.ShapeDtypeStruct((1, c), jnp.float32),
    )(x)

    out = pl.pallas_call(
        functools.partial(_replace_kernel, inv_n=1.0 / rows),
        grid=(nblk,),
        in_specs=[
            pl.BlockSpec((_BLOCK_ROWS, c), lambda i: (i, 0)),
            pl.BlockSpec((1, c), lambda i: (0, 0)),
        ],
        out_specs=pl.BlockSpec((_BLOCK_ROWS, c), lambda i: (i, 0)),
        out_shape=jax.ShapeDtypeStruct((rows, c), jnp.float32),
    )(x, sums)

    return out.reshape(orig_shape)


# speedup vs baseline: 6.5450x; 1.4861x over previous
import functools

import jax
import jax.numpy as jnp
from jax.experimental import pallas as pl
from jax.experimental.pallas import tpu as pltpu

_BLOCK_ROWS = 512


def _two_phase_kernel(x_ref, o_ref, acc_ref, cache_ref, *, nblk, inv_n):
    i = pl.program_id(0)

    @pl.when(i == 0)
    def _init():
        acc_ref[...] = jnp.zeros_like(acc_ref)

    @pl.when(i < nblk)
    def _phase1():
        x = x_ref[...]
        acc_ref[...] += jnp.sum(x, axis=0, keepdims=True)
        cache_ref[pl.ds(i * _BLOCK_ROWS, _BLOCK_ROWS), :] = x.astype(jnp.bfloat16)

    @pl.when(i >= nblk)
    def _phase2():
        j = i - nblk
        mean = acc_ref[...] * inv_n
        xc = cache_ref[pl.ds(j * _BLOCK_ROWS, _BLOCK_ROWS), :].astype(jnp.float32)
        lane = jax.lax.broadcasted_iota(jnp.int32, xc.shape, dimension=1)
        o_ref[...] = jnp.where(lane % 2 == 0, jnp.broadcast_to(mean, xc.shape), xc)


def kernel(inputs):
    orig_shape = inputs.shape
    c = orig_shape[-1]
    rows = 1
    for d in orig_shape[:-1]:
        rows *= d
    x = inputs.reshape(rows, c)
    nblk = rows // _BLOCK_ROWS

    out = pl.pallas_call(
        functools.partial(_two_phase_kernel, nblk=nblk, inv_n=1.0 / rows),
        grid=(2 * nblk,),
        in_specs=[
            pl.BlockSpec((_BLOCK_ROWS, c), lambda i: (jnp.minimum(i, nblk - 1), 0)),
        ],
        out_specs=pl.BlockSpec(
            (_BLOCK_ROWS, c), lambda i: (jnp.maximum(i - nblk, 0), 0)
        ),
        out_shape=jax.ShapeDtypeStruct((rows, c), jnp.float32),
        scratch_shapes=[
            pltpu.VMEM((1, c), jnp.float32),
            pltpu.VMEM((rows, c), jnp.bfloat16),
        ],
    )(x)

    return out.reshape(orig_shape)


# speedup vs baseline: 6.5712x; 1.0040x over previous
import functools

import jax
import jax.numpy as jnp
from jax.experimental import pallas as pl

_STRIPE_W = 256


def _stripe_kernel(x_ref, o_ref, *, inv_n):
    x = x_ref[...]
    mean = jnp.sum(x, axis=0, keepdims=True) * inv_n
    lane = jax.lax.broadcasted_iota(jnp.int32, x.shape, dimension=1)
    o_ref[...] = jnp.where(lane % 2 == 0, jnp.broadcast_to(mean, x.shape), x)


def kernel(inputs):
    orig_shape = inputs.shape
    c = orig_shape[-1]
    rows = 1
    for d in orig_shape[:-1]:
        rows *= d
    x = inputs.reshape(rows, c)
    nstripes = c // _STRIPE_W

    out = pl.pallas_call(
        functools.partial(_stripe_kernel, inv_n=1.0 / rows),
        grid=(nstripes,),
        in_specs=[pl.BlockSpec((rows, _STRIPE_W), lambda s: (0, s))],
        out_specs=pl.BlockSpec((rows, _STRIPE_W), lambda s: (0, s)),
        out_shape=jax.ShapeDtypeStruct((rows, c), jnp.float32),
    )(x)

    return out.reshape(orig_shape)
